# baseline (speedup 1.0000x reference)
import functools

import jax
import jax.numpy as jnp
from jax import lax
from jax.experimental import pallas as pl
from jax.experimental.pallas import tpu as pltpu
from jax.experimental.pallas import tpu_sc as plsc

D = 256
HD = 128
CHUNK = 160


def _rtne_bf16_bits(u):
    return u + jnp.uint32(0x7FFF) + ((u >> 16) & jnp.uint32(1))


def _cast_body(e_ref, pk_ref):
    x = e_ref[...]
    ul = jax.lax.bitcast_convert_type(x[:, :HD], jnp.uint32)
    uh = jax.lax.bitcast_convert_type(x[:, HD:], jnp.uint32)
    lo = _rtne_bf16_bits(ul) >> 16
    hi = _rtne_bf16_bits(uh) & jnp.uint32(0xFFFF0000)
    pk_ref[...] = jax.lax.bitcast_convert_type(lo | hi, jnp.int32)


@functools.lru_cache(maxsize=None)
def _make_cast(V: int, RB: int):
    nb = V // RB
    assert nb * RB == V
    return pl.pallas_call(
        _cast_body,
        grid=(nb,),
        in_specs=[pl.BlockSpec((RB, D), lambda i: (i, 0))],
        out_specs=pl.BlockSpec((RB, HD), lambda i: (i, 0)),
        out_shape=jax.ShapeDtypeStruct((V, HD), jnp.int32),
    )


@functools.lru_cache(maxsize=None)
def _make_gather2(B: int, dtype=jnp.int32, width=HD):
    info = plsc.get_sparse_core_info()
    nc, ns = info.num_cores, info.num_subcores
    nw = nc * ns
    nchunk = B // CHUNK
    assert nchunk * CHUNK == B
    units_per_worker = -(-nchunk // nw)
    mesh = plsc.VectorSubcoreMesh(core_axis_name="c", subcore_axis_name="s")

    @functools.partial(
        pl.kernel,
        mesh=mesh,
        out_type=[
            jax.ShapeDtypeStruct((B, width), dtype),
            jax.ShapeDtypeStruct((B, width), dtype),
        ],
        scratch_types=[
            pltpu.VMEM((CHUNK,), jnp.int32),
            pltpu.VMEM((CHUNK,), jnp.int32),
            pltpu.VMEM((CHUNK, width), dtype),
            pltpu.VMEM((CHUNK, width), dtype),
            pltpu.SemaphoreType.DMA,
            pltpu.SemaphoreType.DMA,
        ],
    )
    def gather2(uids_hbm, iids_hbm, eg_hbm, ed_hbm, outu_hbm, outi_hbm,
                uidx_v, iidx_v, urow_v, irow_v, usem, isem):
        wid = lax.axis_index("s") * nc + lax.axis_index("c")

        def body(k, carry):
            g = wid * units_per_worker + k

            @pl.when(g < nchunk)
            def _():
                base = g * CHUNK
                pltpu.sync_copy(uids_hbm.at[pl.ds(base, CHUNK)], uidx_v)
                pltpu.sync_copy(iids_hbm.at[pl.ds(base, CHUNK)], iidx_v)
                ucp = pltpu.async_copy(eg_hbm.at[uidx_v], urow_v, usem)
                icp = pltpu.async_copy(ed_hbm.at[iidx_v], irow_v, isem)
                ucp.wait()
                pltpu.sync_copy(urow_v, outu_hbm.at[pl.ds(base, CHUNK)])
                icp.wait()
                pltpu.sync_copy(irow_v, outi_hbm.at[pl.ds(base, CHUNK)])

            return carry

        lax.fori_loop(0, units_per_worker, body, 0)

    return gather2


def _unpack_bf16(p_i32):
    u = jax.lax.bitcast_convert_type(p_i32, jnp.uint32)
    lo = jax.lax.bitcast_convert_type(u << 16, jnp.float32)
    hi = jax.lax.bitcast_convert_type(u & jnp.uint32(0xFFFF0000), jnp.float32)
    return lo.astype(jnp.bfloat16), hi.astype(jnp.bfloat16)


def _mlp_core(u_ref, i_ref, w1a_ref, w1b_ref, b1_ref, w2_ref, b2_ref,
              w3t_ref, b3_ref, out_ref):
    bf = jnp.bfloat16
    u = jnp.concatenate(_unpack_bf16(u_ref[...]), axis=1)
    i = jnp.concatenate(_unpack_bf16(i_ref[...]), axis=1)
    h = (
        jnp.dot(u, w1a_ref[...].astype(bf), preferred_element_type=jnp.float32)
        + jnp.dot(i, w1b_ref[...].astype(bf), preferred_element_type=jnp.float32)
        + b1_ref[...]
    )
    h = jnp.maximum(h, 0.0)
    h = jnp.dot(h.astype(bf), w2_ref[...].astype(bf),
                preferred_element_type=jnp.float32) + b2_ref[...]
    h = jnp.maximum(h, 0.0)
    out_ref[...] = (
        lax.dot_general(w3t_ref[...].astype(bf), h.astype(bf),
                        (((1,), (1,)), ((), ())),
                        preferred_element_type=jnp.float32)
        + b3_ref[...]
    )[None]


def _mlp_copy_body(u_ref, i_ref, w1a_ref, w1b_ref, b1_ref, w2_ref, b2_ref,
                   w3t_ref, b3_ref, egs_ref, eds_ref, *rest, s_off, cb, vmax,
                   aliased):
    if aliased:
        (_ega, _eda, out_ref, ego_ref, edo_ref, semg, semd) = rest
    else:
        (out_ref, ego_ref, edo_ref, semg, semd) = rest
    base = (s_off + pl.program_id(0)) * cb
    base = pl.multiple_of(jnp.minimum(base, vmax), 8)
    cg = pltpu.make_async_copy(egs_ref.at[pl.ds(base, cb)],
                               ego_ref.at[pl.ds(base, cb)], semg)
    cd = pltpu.make_async_copy(eds_ref.at[pl.ds(base, cb)],
                               edo_ref.at[pl.ds(base, cb)], semd)
    cg.start()
    cd.start()
    _mlp_core(u_ref, i_ref, w1a_ref, w1b_ref, b1_ref, w2_ref, b2_ref,
              w3t_ref, b3_ref, out_ref)
    cg.wait()
    cd.wait()


@functools.lru_cache(maxsize=None)
def _make_mlp(B: int, R: int, V: int, s: int, nstage: int,
              interpret: bool = False):
    nb = B // R
    assert nb * R == B
    cb = -(-V // (nstage * nb))
    cb = -(-cb // 8) * 8
    assert cb * nstage * nb >= V and (V - cb) % 8 == 0
    rep = lambda i: (0, 0)
    hbm = pl.BlockSpec(memory_space=pltpu.MemorySpace.HBM)
    aliased = s > 0
    in_specs = [
        pl.BlockSpec((R, HD), lambda i: (i, 0)),
        pl.BlockSpec((R, HD), lambda i: (i, 0)),
        pl.BlockSpec((D, D), rep),
        pl.BlockSpec((D, D), rep),
        pl.BlockSpec((1, D), rep),
        pl.BlockSpec((D, D), rep),
        pl.BlockSpec((1, D), rep),
        pl.BlockSpec((1, D), rep),
        pl.BlockSpec((1, 1), rep),
        hbm,
        hbm,
    ]
    if aliased:
        in_specs += [hbm, hbm]
    return pl.pallas_call(
        functools.partial(_mlp_copy_body, s_off=s * nb, cb=cb, vmax=V - cb,
                          aliased=aliased),
        grid=(nb,),
        in_specs=in_specs,
        out_specs=[pl.BlockSpec((1, 1, R), lambda i: (i, 0, 0)), hbm, hbm],
        out_shape=[jax.ShapeDtypeStruct((nb, 1, R), jnp.float32),
                   jax.ShapeDtypeStruct((V, D), jnp.float32),
                   jax.ShapeDtypeStruct((V, D), jnp.float32)],
        input_output_aliases={11: 1, 12: 2} if aliased else {},
        scratch_shapes=[pltpu.SemaphoreType.DMA, pltpu.SemaphoreType.DMA],
        interpret=interpret,
    )


def kernel(uids, iids, E_g, E_d, W1, b1, W2, b2, W3, b3):
    B = uids.shape[0]
    S = 5
    Bs = B // S
    assert Bs * S == B
    uids = uids.astype(jnp.int32)
    iids = iids.astype(jnp.int32)
    V = E_g.shape[0]
    cast = _make_cast(V, 2000)
    eg_pk = cast(E_g)
    ed_pk = cast(E_d)
    gather_pk = _make_gather2(Bs)
    R = 5000
    w_args = (W1[:D], W1[D:], b1.reshape(1, D), W2, b2.reshape(1, D),
              W3.reshape(1, D), b3.reshape(1, 1))
    preds = []
    eg_cp = ed_cp = None
    for s in range(S):
        sl = slice(s * Bs, (s + 1) * Bs)
        u_emb, i_emb = gather_pk(uids[sl], iids[sl], eg_pk, ed_pk)
        mlp = _make_mlp(Bs, R, V, s, S)
        extra = (E_g, E_d) if s == 0 else (E_g, E_d, eg_cp, ed_cp)
        out, eg_cp, ed_cp = mlp(u_emb, i_emb, *w_args, *extra)
        preds.append(out.reshape(1, Bs))
    return (jnp.concatenate(preds, axis=1), eg_cp, ed_cp)


# speedup vs baseline: 11.2735x; 11.2735x over previous
import functools

import jax
import jax.numpy as jnp
from jax import lax
from jax.experimental import pallas as pl
from jax.experimental.pallas import tpu as pltpu
from jax.experimental.pallas import tpu_sc as plsc

D = 256
HD = 128
CHUNK = 160


def _rtne_bf16_bits(u):
    return u + jnp.uint32(0x7FFF) + ((u >> 16) & jnp.uint32(1))


def _cast_body(e_ref, pk_ref):
    x = e_ref[...]
    ul = jax.lax.bitcast_convert_type(x[:, :HD], jnp.uint32)
    uh = jax.lax.bitcast_convert_type(x[:, HD:], jnp.uint32)
    lo = _rtne_bf16_bits(ul) >> 16
    hi = _rtne_bf16_bits(uh) & jnp.uint32(0xFFFF0000)
    pk_ref[...] = jax.lax.bitcast_convert_type(lo | hi, jnp.int32)


@functools.lru_cache(maxsize=None)
def _make_cast(V: int, RB: int):
    nb = V // RB
    assert nb * RB == V
    return pl.pallas_call(
        _cast_body,
        grid=(nb,),
        in_specs=[pl.BlockSpec((RB, D), lambda i: (i, 0))],
        out_specs=pl.BlockSpec((RB, HD), lambda i: (i, 0)),
        out_shape=jax.ShapeDtypeStruct((V, HD), jnp.int32),
    )


@functools.lru_cache(maxsize=None)
def _make_gather2(B: int, dtype=jnp.int32, width=HD):
    info = plsc.get_sparse_core_info()
    nc, ns = info.num_cores, info.num_subcores
    nw = nc * ns
    nchunk = B // CHUNK
    assert nchunk * CHUNK == B
    units_per_worker = -(-nchunk // nw)
    mesh = plsc.VectorSubcoreMesh(core_axis_name="c", subcore_axis_name="s")

    @functools.partial(
        pl.kernel,
        mesh=mesh,
        out_type=[
            jax.ShapeDtypeStruct((B, width), dtype),
            jax.ShapeDtypeStruct((B, width), dtype),
        ],
        scratch_types=[
            pltpu.VMEM((CHUNK,), jnp.int32),
            pltpu.VMEM((CHUNK,), jnp.int32),
            pltpu.VMEM((CHUNK, width), dtype),
            pltpu.VMEM((CHUNK, width), dtype),
            pltpu.SemaphoreType.DMA,
            pltpu.SemaphoreType.DMA,
        ],
    )
    def gather2(uids_hbm, iids_hbm, eg_hbm, ed_hbm, outu_hbm, outi_hbm,
                uidx_v, iidx_v, urow_v, irow_v, usem, isem):
        wid = lax.axis_index("s") * nc + lax.axis_index("c")

        def body(k, carry):
            g = wid * units_per_worker + k

            @pl.when(g < nchunk)
            def _():
                base = g * CHUNK
                pltpu.sync_copy(uids_hbm.at[pl.ds(base, CHUNK)], uidx_v)
                pltpu.sync_copy(iids_hbm.at[pl.ds(base, CHUNK)], iidx_v)
                ucp = pltpu.async_copy(eg_hbm.at[uidx_v], urow_v, usem)
                icp = pltpu.async_copy(ed_hbm.at[iidx_v], irow_v, isem)
                ucp.wait()
                pltpu.sync_copy(urow_v, outu_hbm.at[pl.ds(base, CHUNK)])
                icp.wait()
                pltpu.sync_copy(irow_v, outi_hbm.at[pl.ds(base, CHUNK)])

            return carry

        lax.fori_loop(0, units_per_worker, body, 0)

    return gather2


def _unpack_bf16(p_i32):
    u = jax.lax.bitcast_convert_type(p_i32, jnp.uint32)
    lo = jax.lax.bitcast_convert_type(u << 16, jnp.float32)
    hi = jax.lax.bitcast_convert_type(u & jnp.uint32(0xFFFF0000), jnp.float32)
    return lo.astype(jnp.bfloat16), hi.astype(jnp.bfloat16)


def _mlp_core(u_ref, i_ref, w1a_ref, w1b_ref, b1_ref, w2_ref, b2_ref,
              w3t_ref, b3_ref, out_ref):
    bf = jnp.bfloat16
    u = jnp.concatenate(_unpack_bf16(u_ref[...]), axis=1)
    i = jnp.concatenate(_unpack_bf16(i_ref[...]), axis=1)
    h = (
        jnp.dot(u, w1a_ref[...].astype(bf), preferred_element_type=jnp.float32)
        + jnp.dot(i, w1b_ref[...].astype(bf), preferred_element_type=jnp.float32)
        + b1_ref[...]
    )
    h = jnp.maximum(h, 0.0)
    h = jnp.dot(h.astype(bf), w2_ref[...].astype(bf),
                preferred_element_type=jnp.float32) + b2_ref[...]
    h = jnp.maximum(h, 0.0)
    out_ref[...] = (
        lax.dot_general(w3t_ref[...].astype(bf), h.astype(bf),
                        (((1,), (1,)), ((), ())),
                        preferred_element_type=jnp.float32)
        + b3_ref[...]
    )[None]


def _mlp_copy_body(u_ref, i_ref, w1a_ref, w1b_ref, b1_ref, w2_ref, b2_ref,
                   w3t_ref, b3_ref, egs_ref, eds_ref, *rest, aliased):
    if aliased:
        (_ega, _eda, out_ref, ego_ref, edo_ref) = rest
    else:
        (out_ref, ego_ref, edo_ref) = rest
    ego_ref[...] = egs_ref[...]
    edo_ref[...] = eds_ref[...]
    _mlp_core(u_ref, i_ref, w1a_ref, w1b_ref, b1_ref, w2_ref, b2_ref,
              w3t_ref, b3_ref, out_ref)


@functools.lru_cache(maxsize=None)
def _make_mlp(B: int, R: int, V: int, s: int, nstage: int,
              interpret: bool = False):
    nb = B // R
    assert nb * R == B
    cb = V // (nstage * nb)
    assert cb * nstage * nb == V and cb % 8 == 0
    rep = lambda i: (0, 0)
    s_off = s * nb
    cmap = lambda i: (s_off + i, 0)
    hbm = pl.BlockSpec(memory_space=pltpu.MemorySpace.HBM)
    aliased = s > 0
    in_specs = [
        pl.BlockSpec((R, HD), lambda i: (i, 0)),
        pl.BlockSpec((R, HD), lambda i: (i, 0)),
        pl.BlockSpec((D, D), rep),
        pl.BlockSpec((D, D), rep),
        pl.BlockSpec((1, D), rep),
        pl.BlockSpec((D, D), rep),
        pl.BlockSpec((1, D), rep),
        pl.BlockSpec((1, D), rep),
        pl.BlockSpec((1, 1), rep),
        pl.BlockSpec((cb, D), cmap),
        pl.BlockSpec((cb, D), cmap),
    ]
    if aliased:
        in_specs += [hbm, hbm]
    return pl.pallas_call(
        functools.partial(_mlp_copy_body, aliased=aliased),
        grid=(nb,),
        in_specs=in_specs,
        out_specs=[pl.BlockSpec((1, 1, R), lambda i: (i, 0, 0)),
                   pl.BlockSpec((cb, D), cmap),
                   pl.BlockSpec((cb, D), cmap)],
        out_shape=[jax.ShapeDtypeStruct((nb, 1, R), jnp.float32),
                   jax.ShapeDtypeStruct((V, D), jnp.float32),
                   jax.ShapeDtypeStruct((V, D), jnp.float32)],
        input_output_aliases={11: 1, 12: 2} if aliased else {},
        interpret=interpret,
    )


def kernel(uids, iids, E_g, E_d, W1, b1, W2, b2, W3, b3):
    B = uids.shape[0]
    S = 5
    Bs = B // S
    assert Bs * S == B
    uids = uids.astype(jnp.int32)
    iids = iids.astype(jnp.int32)
    V = E_g.shape[0]
    cast = _make_cast(V, 2000)
    eg_pk = cast(E_g)
    ed_pk = cast(E_d)
    gather_pk = _make_gather2(Bs)
    R = 4000
    w_args = (W1[:D], W1[D:], b1.reshape(1, D), W2, b2.reshape(1, D),
              W3.reshape(1, D), b3.reshape(1, 1))
    preds = []
    eg_cp = ed_cp = None
    for s in range(S):
        sl = slice(s * Bs, (s + 1) * Bs)
        u_emb, i_emb = gather_pk(uids[sl], iids[sl], eg_pk, ed_pk)
        mlp = _make_mlp(Bs, R, V, s, S)
        extra = (E_g, E_d) if s == 0 else (E_g, E_d, eg_cp, ed_cp)
        out, eg_cp, ed_cp = mlp(u_emb, i_emb, *w_args, *extra)
        preds.append(out.reshape(1, Bs))
    return (jnp.concatenate(preds, axis=1), eg_cp, ed_cp)


# speedup vs baseline: 12.0257x; 1.0667x over previous
import functools

import jax
import jax.numpy as jnp
from jax import lax
from jax.experimental import pallas as pl
from jax.experimental.pallas import tpu as pltpu
from jax.experimental.pallas import tpu_sc as plsc

D = 256
HD = 128
CHUNK = 160


def _rtne_bf16_bits(u):
    return u + jnp.uint32(0x7FFF) + ((u >> 16) & jnp.uint32(1))


def _castcopy_body(e_ref, cp_ref, pk_ref):
    x = e_ref[...]
    cp_ref[...] = x
    ul = jax.lax.bitcast_convert_type(x[:, :HD], jnp.uint32)
    uh = jax.lax.bitcast_convert_type(x[:, HD:], jnp.uint32)
    lo = _rtne_bf16_bits(ul) >> 16
    hi = _rtne_bf16_bits(uh) & jnp.uint32(0xFFFF0000)
    pk_ref[...] = jax.lax.bitcast_convert_type(lo | hi, jnp.int32)


@functools.lru_cache(maxsize=None)
def _make_castcopy(V: int, RB: int):
    nb = V // RB
    assert nb * RB == V
    return pl.pallas_call(
        _castcopy_body,
        grid=(nb,),
        in_specs=[pl.BlockSpec((RB, D), lambda i: (i, 0))],
        out_specs=[pl.BlockSpec((RB, D), lambda i: (i, 0)),
                   pl.BlockSpec((RB, HD), lambda i: (i, 0))],
        out_shape=[jax.ShapeDtypeStruct((V, D), jnp.float32),
                   jax.ShapeDtypeStruct((V, HD), jnp.int32)],
    )


@functools.lru_cache(maxsize=None)
def _make_gather2(B: int, dtype=jnp.int32, width=HD):
    info = plsc.get_sparse_core_info()
    nc, ns = info.num_cores, info.num_subcores
    nw = nc * ns
    nchunk = B // CHUNK
    assert nchunk * CHUNK == B
    units_per_worker = -(-nchunk // nw)
    mesh = plsc.VectorSubcoreMesh(core_axis_name="c", subcore_axis_name="s")

    @functools.partial(
        pl.kernel,
        mesh=mesh,
        out_type=[
            jax.ShapeDtypeStruct((B, width), dtype),
            jax.ShapeDtypeStruct((B, width), dtype),
        ],
        scratch_types=[
            pltpu.VMEM((CHUNK,), jnp.int32),
            pltpu.VMEM((CHUNK,), jnp.int32),
            pltpu.VMEM((CHUNK, width), dtype),
            pltpu.VMEM((CHUNK, width), dtype),
            pltpu.SemaphoreType.DMA,
            pltpu.SemaphoreType.DMA,
        ],
    )
    def gather2(uids_hbm, iids_hbm, eg_hbm, ed_hbm, outu_hbm, outi_hbm,
                uidx_v, iidx_v, urow_v, irow_v, usem, isem):
        wid = lax.axis_index("s") * nc + lax.axis_index("c")

        def body(k, carry):
            g = wid * units_per_worker + k

            @pl.when(g < nchunk)
            def _():
                base = g * CHUNK
                pltpu.sync_copy(uids_hbm.at[pl.ds(base, CHUNK)], uidx_v)
                pltpu.sync_copy(iids_hbm.at[pl.ds(base, CHUNK)], iidx_v)
                ucp = pltpu.async_copy(eg_hbm.at[uidx_v], urow_v, usem)
                icp = pltpu.async_copy(ed_hbm.at[iidx_v], irow_v, isem)
                ucp.wait()
                pltpu.sync_copy(urow_v, outu_hbm.at[pl.ds(base, CHUNK)])
                icp.wait()
                pltpu.sync_copy(irow_v, outi_hbm.at[pl.ds(base, CHUNK)])

            return carry

        lax.fori_loop(0, units_per_worker, body, 0)

    return gather2


def _unpack_bf16(p_i32):
    u = jax.lax.bitcast_convert_type(p_i32, jnp.uint32)
    lo = jax.lax.bitcast_convert_type(u << 16, jnp.float32)
    hi = jax.lax.bitcast_convert_type(u & jnp.uint32(0xFFFF0000), jnp.float32)
    return lo.astype(jnp.bfloat16), hi.astype(jnp.bfloat16)


def _mlp_core(u_ref, i_ref, w1a_ref, w1b_ref, b1_ref, w2_ref, b2_ref,
              w3t_ref, b3_ref, out_ref):
    bf = jnp.bfloat16
    u = jnp.concatenate(_unpack_bf16(u_ref[...]), axis=1)
    i = jnp.concatenate(_unpack_bf16(i_ref[...]), axis=1)
    h = (
        jnp.dot(u, w1a_ref[...].astype(bf), preferred_element_type=jnp.float32)
        + jnp.dot(i, w1b_ref[...].astype(bf), preferred_element_type=jnp.float32)
        + b1_ref[...]
    )
    h = jnp.maximum(h, 0.0)
    h = jnp.dot(h.astype(bf), w2_ref[...].astype(bf),
                preferred_element_type=jnp.float32) + b2_ref[...]
    h = jnp.maximum(h, 0.0)
    out_ref[...] = (
        lax.dot_general(w3t_ref[...].astype(bf), h.astype(bf),
                        (((1,), (1,)), ((), ())),
                        preferred_element_type=jnp.float32)
        + b3_ref[...]
    )[None]


@functools.lru_cache(maxsize=None)
def _make_mlp(B: int, R: int, interpret: bool = False):
    nb = B // R
    assert nb * R == B
    rep = lambda i: (0, 0)
    return pl.pallas_call(
        _mlp_core,
        grid=(nb,),
        in_specs=[
            pl.BlockSpec((R, HD), lambda i: (i, 0)),
            pl.BlockSpec((R, HD), lambda i: (i, 0)),
            pl.BlockSpec((D, D), rep),
            pl.BlockSpec((D, D), rep),
            pl.BlockSpec((1, D), rep),
            pl.BlockSpec((D, D), rep),
            pl.BlockSpec((1, D), rep),
            pl.BlockSpec((1, D), rep),
            pl.BlockSpec((1, 1), rep),
        ],
        out_specs=pl.BlockSpec((1, 1, R), lambda i: (i, 0, 0)),
        out_shape=jax.ShapeDtypeStruct((nb, 1, R), jnp.float32),
        interpret=interpret,
    )


def kernel(uids, iids, E_g, E_d, W1, b1, W2, b2, W3, b3):
    B = uids.shape[0]
    S = 5
    Bs = B // S
    assert Bs * S == B
    uids = uids.astype(jnp.int32)
    iids = iids.astype(jnp.int32)
    V = E_g.shape[0]
    castcopy = _make_castcopy(V, 2000)
    eg_cp, eg_pk = castcopy(E_g)
    ed_cp, ed_pk = castcopy(E_d)
    gather_pk = _make_gather2(Bs)
    mlp = _make_mlp(Bs, 5000)
    w_args = (W1[:D], W1[D:], b1.reshape(1, D), W2, b2.reshape(1, D),
              W3.reshape(1, D), b3.reshape(1, 1))
    preds = []
    for s in range(S):
        sl = slice(s * Bs, (s + 1) * Bs)
        u_emb, i_emb = gather_pk(uids[sl], iids[sl], eg_pk, ed_pk)
        out = mlp(u_emb, i_emb, *w_args)
        preds.append(out.reshape(1, Bs))
    return (jnp.concatenate(preds, axis=1), eg_cp, ed_cp)
